# baseline (speedup 1.0000x reference)
import jax
import jax.numpy as jnp
from jax.experimental import pallas as pl
from jax.experimental.pallas import tpu as pltpu

N = 48
P = N * N
NCL = 32
NDL = 16
QT = 384
NQT = P // QT
NROWS = 40
GID_OFF = 6_000_000


def _loss_kernel(consts_ref, cw_ref, dw_ref, gw_ref, out_ref):
    qi = pl.program_id(0)
    pi = pl.program_id(1)

    @pl.when((qi == 0) & (pi == 0))
    def _init():
        out_ref[0, 0] = 0.0

    @pl.when(pi <= qi)
    def _active():
        dt = dw_ref[pl.ds(pi * QT, QT), :]
        ma = jnp.maximum(jnp.max(dt, axis=1, keepdims=True), 0.0)
        ea = jnp.concatenate([jnp.exp(dt - ma), jnp.exp(-ma)], axis=1)

        gt = gw_ref[pl.ds(qi * QT, QT), :]
        mb = jnp.maximum(jnp.max(gt, axis=1, keepdims=True), 0.0)
        eb = jnp.concatenate([jnp.exp(gt - mb), jnp.exp(-mb)], axis=1)

        m = jax.lax.dot_general(
            ea, eb, (((1,), (1,)), ((), ())),
            preferred_element_type=jnp.float32,
            precision=jax.lax.Precision.DEFAULT,
        )

        qr = jax.lax.broadcasted_iota(jnp.int32, (1, QT), 1) + qi * QT
        kqr = qr // N
        lqr = qr - kqr * N
        pv = jax.lax.broadcasted_iota(jnp.int32, (QT, 1), 0) + pi * QT
        ipc = pv // N
        jpc = pv - ipc * N
        valid = (jpc > lqr) & (kqr <= lqr) & ((ipc < kqr) | (pi < qi))
        tile_sum = jnp.sum(jnp.where(valid, jnp.log(m), 0.0))

        extra = tile_sum

        @pl.when(pi == 0)
        def _cols():
            qc = jax.lax.broadcasted_iota(jnp.int32, (QT, 1), 0) + qi * QT
            kqc = qc // N
            lqc = qc - kqc * N
            ccnt = jnp.where(kqc <= lqc, kqc * (47 - lqc), 0).astype(jnp.float32)
            out_ref[0, 0] += jnp.sum(mb * ccnt)

        @pl.when((qi == 0) & (pi == 0))
        def _first_step():
            pva = jax.lax.broadcasted_iota(jnp.int32, (P, 1), 0)
            ipa = pva // N
            jpa = pva - ipa * N
            dwa = dw_ref[...]
            maa = jnp.maximum(jnp.max(dwa, axis=1, keepdims=True), 0.0)
            mrow = jnp.minimum(jpa - 1, 46) - ipa
            rcnt = jnp.where(mrow > 0, mrow * (mrow + 1) // 2, 0).astype(
                jnp.float32
            )
            row_term = jnp.sum(maa * rcnt)

            cw = cw_ref[...]
            mc = jnp.maximum(jnp.max(cw, axis=1, keepdims=True), 0.0)
            lse_c = mc + jnp.log(
                jnp.sum(jnp.exp(cw - mc), axis=1, keepdims=True) + jnp.exp(-mc)
            )
            cont_sum = jnp.sum(jnp.where(ipa <= jpa, lse_c, 0.0))

            labs, iis, kks, lls, jjs, gids, isc = [], [], [], [], [], [], []
            for r in range(NROWS):
                lab = consts_ref[r, 0]
                i = consts_ref[r, 1]
                k = consts_ref[r, 2]
                l = consts_ref[r, 3]
                j = consts_ref[r, 4]
                cont = k < 0
                gid = jnp.where(
                    cont,
                    i * N + j,
                    ((i * N + k) * N + l) * N + j + GID_OFF,
                )
                labs.append(lab); iis.append(i); kks.append(k)
                lls.append(l); jjs.append(j); gids.append(gid); isc.append(cont)

            oh32 = jax.lax.broadcasted_iota(jnp.int32, (1, NCL), 1)
            oh16 = jax.lax.broadcasted_iota(jnp.int32, (1, NDL), 1)
            corr = jnp.float32(0.0)
            for r in range(NROWS):
                last = jnp.bool_(True)
                for r2 in range(r + 1, NROWS):
                    last = jnp.logical_and(last, gids[r] != gids[r2])
                pidx = iis[r] * N + jjs[r]
                qidx = jnp.where(isc[r], 0, (kks[r] + 1) * N + (lls[r] - 1))
                sel32 = (oh32 == labs[r]).astype(jnp.float32)
                sel16 = (oh16 == labs[r]).astype(jnp.float32)
                vc = jnp.sum(cw_ref[pl.ds(pidx, 1), :] * sel32)
                vd = jnp.sum(
                    (dw_ref[pl.ds(pidx, 1), :] + gw_ref[pl.ds(qidx, 1), :])
                    * sel16
                )
                val = jnp.where(isc[r], vc, vd)
                corr = corr + jnp.where(last, val, 0.0)

            out_ref[0, 0] += row_term + cont_sum - corr

        out_ref[0, 0] += extra


@jax.jit
def kernel(cont_weights, disc_weights, gap_weights, constituents):
    cw = cont_weights[0].reshape(P, NCL)
    dw = disc_weights[0].reshape(P, NDL)
    gw = gap_weights[0].reshape(P, NDL)
    consts = constituents.astype(jnp.int32)

    out = pl.pallas_call(
        _loss_kernel,
        grid=(NQT, NQT),
        in_specs=[
            pl.BlockSpec(memory_space=pltpu.SMEM),
            pl.BlockSpec((P, NCL), lambda qi, pi: (0, 0)),
            pl.BlockSpec((P, NDL), lambda qi, pi: (0, 0)),
            pl.BlockSpec((P, NDL), lambda qi, pi: (0, 0)),
        ],
        out_specs=pl.BlockSpec(
            (1, 1), lambda qi, pi: (0, 0), memory_space=pltpu.SMEM
        ),
        out_shape=jax.ShapeDtypeStruct((1, 1), jnp.float32),
        compiler_params=pltpu.CompilerParams(
            dimension_semantics=("arbitrary", "arbitrary"),
        ),
    )(consts, cw, dw, gw)
    return out.reshape(1)


# speedup vs baseline: 1.3219x; 1.3219x over previous
import jax
import jax.numpy as jnp
from jax.experimental import pallas as pl
from jax.experimental.pallas import tpu as pltpu

N = 48
P = N * N
NCL = 32
NDL = 16
QT = 768
NQT = P // QT
NROWS = 40
GID_OFF = 6_000_000


def _loss_kernel(consts_ref, cw_ref, dw_ref, gw_ref, out_ref):
    qi = pl.program_id(0)

    @pl.when(qi == 0)
    def _init():
        out_ref[0, 0] = 0.0

    if True:
        dw = dw_ref[...]
        ma = jnp.maximum(jnp.max(dw, axis=1, keepdims=True), 0.0)
        ea = jnp.concatenate([jnp.exp(dw - ma), jnp.exp(-ma)], axis=1)

        gt = gw_ref[pl.ds(qi * QT, QT), :]
        mb = jnp.maximum(jnp.max(gt, axis=1, keepdims=True), 0.0)
        eb = jnp.concatenate([jnp.exp(gt - mb), jnp.exp(-mb)], axis=1)

        m = jax.lax.dot_general(
            ea, eb, (((1,), (1,)), ((), ())),
            preferred_element_type=jnp.float32,
            precision=jax.lax.Precision.DEFAULT,
        )

        qr = jax.lax.broadcasted_iota(jnp.int32, (1, QT), 1) + qi * QT
        kqr = qr // N
        lqr = qr - kqr * N
        pv = jax.lax.broadcasted_iota(jnp.int32, (P, 1), 0)
        ipc = pv // N
        jpc = pv - ipc * N
        valid = (ipc < kqr) & (jpc > lqr) & (kqr <= lqr)
        tile_sum = jnp.sum(jnp.where(valid, jnp.log(m), 0.0))

        extra = tile_sum

        qc = jax.lax.broadcasted_iota(jnp.int32, (QT, 1), 0) + qi * QT
        kqc = qc // N
        lqc = qc - kqc * N
        ccnt = jnp.where(kqc <= lqc, kqc * (47 - lqc), 0).astype(jnp.float32)
        out_ref[0, 0] += jnp.sum(mb * ccnt)

        @pl.when(qi == 0)
        def _first_step():
            mrow = jnp.minimum(jpc - 1, 46) - ipc
            rcnt = jnp.where(mrow > 0, mrow * (mrow + 1) // 2, 0).astype(
                jnp.float32
            )
            row_term = jnp.sum(ma * rcnt)

            cw = cw_ref[...]
            mc = jnp.maximum(jnp.max(cw, axis=1, keepdims=True), 0.0)
            lse_c = mc + jnp.log(
                jnp.sum(jnp.exp(cw - mc), axis=1, keepdims=True) + jnp.exp(-mc)
            )
            cont_sum = jnp.sum(jnp.where(ipc <= jpc, lse_c, 0.0))

            labs, iis, kks, lls, jjs, gids, isc = [], [], [], [], [], [], []
            for r in range(NROWS):
                lab = consts_ref[r, 0]
                i = consts_ref[r, 1]
                k = consts_ref[r, 2]
                l = consts_ref[r, 3]
                j = consts_ref[r, 4]
                cont = k < 0
                gid = jnp.where(
                    cont,
                    i * N + j,
                    ((i * N + k) * N + l) * N + j + GID_OFF,
                )
                labs.append(lab); iis.append(i); kks.append(k)
                lls.append(l); jjs.append(j); gids.append(gid); isc.append(cont)

            oh32 = jax.lax.broadcasted_iota(jnp.int32, (1, NCL), 1)
            oh16 = jax.lax.broadcasted_iota(jnp.int32, (1, NDL), 1)
            corr = jnp.float32(0.0)
            for r in range(NROWS):
                last = jnp.bool_(True)
                for r2 in range(r + 1, NROWS):
                    last = jnp.logical_and(last, gids[r] != gids[r2])
                pidx = iis[r] * N + jjs[r]
                qidx = jnp.where(isc[r], 0, (kks[r] + 1) * N + (lls[r] - 1))
                sel32 = (oh32 == labs[r]).astype(jnp.float32)
                sel16 = (oh16 == labs[r]).astype(jnp.float32)
                vc = jnp.sum(cw_ref[pl.ds(pidx, 1), :] * sel32)
                vd = jnp.sum(
                    (dw_ref[pl.ds(pidx, 1), :] + gw_ref[pl.ds(qidx, 1), :])
                    * sel16
                )
                val = jnp.where(isc[r], vc, vd)
                corr = corr + jnp.where(last, val, 0.0)

            out_ref[0, 0] += row_term + cont_sum - corr

    out_ref[0, 0] += extra


@jax.jit
def kernel(cont_weights, disc_weights, gap_weights, constituents):
    cw = cont_weights[0].reshape(P, NCL)
    dw = disc_weights[0].reshape(P, NDL)
    gw = gap_weights[0].reshape(P, NDL)
    consts = constituents.astype(jnp.int32)

    out = pl.pallas_call(
        _loss_kernel,
        grid=(NQT,),
        in_specs=[
            pl.BlockSpec(memory_space=pltpu.SMEM),
            pl.BlockSpec((P, NCL), lambda qi: (0, 0)),
            pl.BlockSpec((P, NDL), lambda qi: (0, 0)),
            pl.BlockSpec((P, NDL), lambda qi: (0, 0)),
        ],
        out_specs=pl.BlockSpec(
            (1, 1), lambda qi: (0, 0), memory_space=pltpu.SMEM
        ),
        out_shape=jax.ShapeDtypeStruct((1, 1), jnp.float32),
        compiler_params=pltpu.CompilerParams(
            dimension_semantics=("arbitrary",),
        ),
    )(consts, cw, dw, gw)
    return out.reshape(1)


# speedup vs baseline: 1.4505x; 1.0973x over previous
import jax
import jax.numpy as jnp
from jax.experimental import pallas as pl
from jax.experimental.pallas import tpu as pltpu

N = 48
P = N * N
NCL = 32
NDL = 16
QT = 768
NQT = P // QT
NROWS = 40
GID_OFF = 6_000_000


def _loss_kernel(consts_ref, cw_ref, dw_ref, gw_ref, out_ref):
    qi = pl.program_id(0)

    @pl.when(qi == 0)
    def _init():
        out_ref[0, 0] = 0.0

    gt = gw_ref[pl.ds(qi * QT, QT), :]
    mb = jnp.maximum(jnp.max(gt, axis=1, keepdims=True), 0.0)
    eb = jnp.concatenate([jnp.exp(gt - mb), jnp.exp(-mb)], axis=1)

    qr = jax.lax.broadcasted_iota(jnp.int32, (1, QT), 1) + qi * QT
    kqr = qr // N
    lqr = qr - kqr * N

    for qv in range(NQT):

        @pl.when(qi == qv)
        def _tile(qv=qv):
            rws = QT * qv + 720
            dt = dw_ref[:rws, :]
            ma = jnp.maximum(jnp.max(dt, axis=1, keepdims=True), 0.0)
            ea = jnp.concatenate([jnp.exp(dt - ma), jnp.exp(-ma)], axis=1)

            m = jax.lax.dot_general(
                ea, eb, (((1,), (1,)), ((), ())),
                preferred_element_type=jnp.float32,
                precision=jax.lax.Precision.DEFAULT,
            )

            pv = jax.lax.broadcasted_iota(jnp.int32, (rws, 1), 0)
            ipc = pv // N
            jpc = pv - ipc * N
            valid = (ipc < kqr) & (jpc > lqr) & (kqr <= lqr)
            out_ref[0, 0] += jnp.sum(jnp.where(valid, jnp.log(m), 0.0))

    if True:
        qc = jax.lax.broadcasted_iota(jnp.int32, (QT, 1), 0) + qi * QT
        kqc = qc // N
        lqc = qc - kqc * N
        ccnt = jnp.where(kqc <= lqc, kqc * (47 - lqc), 0).astype(jnp.float32)
        out_ref[0, 0] += jnp.sum(mb * ccnt)

        @pl.when(qi == 0)
        def _first_step():
            pva = jax.lax.broadcasted_iota(jnp.int32, (P, 1), 0)
            ipa = pva // N
            jpa = pva - ipa * N
            dwa = dw_ref[...]
            maa = jnp.maximum(jnp.max(dwa, axis=1, keepdims=True), 0.0)
            mrow = jnp.minimum(jpa - 1, 46) - ipa
            rcnt = jnp.where(mrow > 0, mrow * (mrow + 1) // 2, 0).astype(
                jnp.float32
            )
            row_term = jnp.sum(maa * rcnt)

            cw = cw_ref[...]
            mc = jnp.maximum(jnp.max(cw, axis=1, keepdims=True), 0.0)
            lse_c = mc + jnp.log(
                jnp.sum(jnp.exp(cw - mc), axis=1, keepdims=True) + jnp.exp(-mc)
            )
            cont_sum = jnp.sum(jnp.where(ipa <= jpa, lse_c, 0.0))

            labs, iis, kks, lls, jjs, gids, isc = [], [], [], [], [], [], []
            for r in range(NROWS):
                lab = consts_ref[r, 0]
                i = consts_ref[r, 1]
                k = consts_ref[r, 2]
                l = consts_ref[r, 3]
                j = consts_ref[r, 4]
                cont = k < 0
                gid = jnp.where(
                    cont,
                    i * N + j,
                    ((i * N + k) * N + l) * N + j + GID_OFF,
                )
                labs.append(lab); iis.append(i); kks.append(k)
                lls.append(l); jjs.append(j); gids.append(gid); isc.append(cont)

            oh32 = jax.lax.broadcasted_iota(jnp.int32, (1, NCL), 1)
            oh16 = jax.lax.broadcasted_iota(jnp.int32, (1, NDL), 1)
            corr = jnp.float32(0.0)
            for r in range(NROWS):
                last = jnp.bool_(True)
                for r2 in range(r + 1, NROWS):
                    last = jnp.logical_and(last, gids[r] != gids[r2])
                pidx = iis[r] * N + jjs[r]
                qidx = jnp.where(isc[r], 0, (kks[r] + 1) * N + (lls[r] - 1))
                sel32 = (oh32 == labs[r]).astype(jnp.float32)
                sel16 = (oh16 == labs[r]).astype(jnp.float32)
                vc = jnp.sum(cw_ref[pl.ds(pidx, 1), :] * sel32)
                vd = jnp.sum(
                    (dw_ref[pl.ds(pidx, 1), :] + gw_ref[pl.ds(qidx, 1), :])
                    * sel16
                )
                val = jnp.where(isc[r], vc, vd)
                corr = corr + jnp.where(last, val, 0.0)

            out_ref[0, 0] += row_term + cont_sum - corr


@jax.jit
def kernel(cont_weights, disc_weights, gap_weights, constituents):
    cw = cont_weights[0].reshape(P, NCL)
    dw = disc_weights[0].reshape(P, NDL)
    gw = gap_weights[0].reshape(P, NDL)
    consts = constituents.astype(jnp.int32)

    out = pl.pallas_call(
        _loss_kernel,
        grid=(NQT,),
        in_specs=[
            pl.BlockSpec(memory_space=pltpu.SMEM),
            pl.BlockSpec((P, NCL), lambda qi: (0, 0)),
            pl.BlockSpec((P, NDL), lambda qi: (0, 0)),
            pl.BlockSpec((P, NDL), lambda qi: (0, 0)),
        ],
        out_specs=pl.BlockSpec(
            (1, 1), lambda qi: (0, 0), memory_space=pltpu.SMEM
        ),
        out_shape=jax.ShapeDtypeStruct((1, 1), jnp.float32),
        compiler_params=pltpu.CompilerParams(
            dimension_semantics=("arbitrary",),
        ),
    )(consts, cw, dw, gw)
    return out.reshape(1)
